# Initial kernel scaffold; baseline (speedup 1.0000x reference)
import jax
import jax.numpy as jnp
from jax.experimental import pallas as pl


def kernel(indices, table):
    raise NotImplementedError("write your pallas kernel here")


# speedup vs baseline: 1.4053x; 1.4053x over previous
import functools

import jax
import jax.numpy as jnp
from jax import lax
from jax.experimental import pallas as pl
from jax.experimental.pallas import tpu as pltpu
from jax.experimental.pallas import tpu_sc as plsc

VOCAB = 10
DIM = 64
LANES = 16
NUM_CORES = 2
NUM_SUBCORES = 16
NUM_WORKERS = NUM_CORES * NUM_SUBCORES

CHUNK = 512
GROUPS = CHUNK // LANES


def _make_sc_lookup(n_rows: int):
  per_w = n_rows // NUM_WORKERS
  n_chunks = per_w // CHUNK
  assert per_w % CHUNK == 0 and n_chunks % 2 == 0

  mesh = plsc.VectorSubcoreMesh(
      core_axis_name="c", subcore_axis_name="s",
      num_cores=NUM_CORES, num_subcores=NUM_SUBCORES)

  @functools.partial(
      pl.kernel,
      out_type=jax.ShapeDtypeStruct((n_rows * DIM,), jnp.float32),
      mesh=mesh,
      compiler_params=pltpu.CompilerParams(needs_layout_passes=False),
      scratch_types=[
          pltpu.VMEM((VOCAB * DIM,), jnp.float32),
          pltpu.VMEM((GROUPS, LANES), jnp.int32),
          pltpu.VMEM((GROUPS, LANES), jnp.int32),
          pltpu.VMEM((CHUNK * DIM,), jnp.float32),
          pltpu.VMEM((CHUNK * DIM,), jnp.float32),
          pltpu.SemaphoreType.DMA,
          pltpu.SemaphoreType.DMA,
          pltpu.SemaphoreType.DMA,
          pltpu.SemaphoreType.DMA,
      ],
  )
  def lookup(idx_hbm, table_hbm, out_hbm, table_v, idx_v0, idx_v1,
             rows_v0, rows_v1, sem_i0, sem_i1, sem_o0, sem_o1):
    idx_v = (idx_v0, idx_v1)
    rows_v = (rows_v0, rows_v1)
    sem_i = (sem_i0, sem_i1)
    sem_o = (sem_o0, sem_o1)
    wid = lax.axis_index("s") * NUM_CORES + lax.axis_index("c")
    wbase = wid * per_w
    gbase = wbase // LANES

    pltpu.sync_copy(table_hbm, table_v)
    lane = lax.iota(jnp.int32, 16)

    def start_in(g, b):
      off = pl.multiple_of(gbase + g * GROUPS, GROUPS)
      pltpu.async_copy(
          idx_hbm.at[pl.ds(off, GROUPS)], idx_v[b], sem_i[b])

    def wait_in(b):
      pltpu.make_async_copy(
          idx_hbm.at[pl.ds(0, GROUPS)], idx_v[b], sem_i[b]).wait()

    def start_out(g, b):
      off = pl.multiple_of((wbase + g * CHUNK) * DIM, CHUNK * DIM)
      pltpu.async_copy(
          rows_v[b], out_hbm.at[pl.ds(off, CHUNK * DIM)], sem_o[b])

    def wait_out(b):
      pltpu.make_async_copy(
          rows_v[b], out_hbm.at[pl.ds(0, CHUNK * DIM)], sem_o[b]).wait()

    def compute(b):
      rv = rows_v[b]
      iv = idx_v[b]

      def group(gi, carry):
        ivec = iv[gi]
        src_base = ivec * DIM
        dst_base = (lane + gi * LANES) * DIM
        for k in range(DIM):
          vals = plsc.load_gather(table_v, [src_base + k])
          plsc.store_scatter(rv, [dst_base + k], vals)
        return carry

      lax.fori_loop(0, GROUPS, group, 0)

    start_in(0, 0)
    start_in(1, 1)

    def step(t, carry):
      for b in range(2):
        g = 2 * t + b
        wait_in(b)

        @pl.when(t > 0)
        def _():
          wait_out(b)

        compute(b)
        start_out(g, b)

        @pl.when(g + 2 < n_chunks)
        def _():
          start_in(g + 2, b)
      return carry

    lax.fori_loop(0, n_chunks // 2, step, 0)
    wait_out(0)
    wait_out(1)

  return lookup


def kernel(indices, table):
  b, s = indices.shape
  n = b * s
  idx2d = indices.reshape(n // LANES, LANES).astype(jnp.int32)
  out = _make_sc_lookup(n)(idx2d, table.reshape(VOCAB * DIM))
  return out.reshape(b, s, DIM)
